# baseline (speedup 1.0000x reference)
import functools

import jax
import jax.numpy as jnp
from jax import lax
from jax.experimental import pallas as pl
from jax.experimental.pallas import tpu as pltpu

B = 16384
EMB = 64
CH = 1024
NSTEP = B // CH


def _tc_gather_kernel(u_smem, i_smem, ut_hbm, it_hbm, ue_ref, ie_ref,
                      usem, isem):
    def body(r, _):
        uu = u_smem[r]
        ii = i_smem[r]
        pltpu.make_async_copy(ut_hbm.at[uu], ue_ref.at[r], usem).start()
        pltpu.make_async_copy(it_hbm.at[ii], ie_ref.at[r], isem).start()
        return _

    lax.fori_loop(0, CH, body, 0, unroll=8)
    pltpu.make_async_copy(ut_hbm.at[pl.ds(0, CH)], ue_ref, usem).wait()
    pltpu.make_async_copy(it_hbm.at[pl.ds(0, CH)], ie_ref, isem).wait()


@jax.jit
def _tc_gather(user_table, item_table, u, i):
    return pl.pallas_call(
        _tc_gather_kernel,
        grid=(NSTEP,),
        in_specs=[
            pl.BlockSpec((CH,), lambda j: (j,), memory_space=pltpu.SMEM),
            pl.BlockSpec((CH,), lambda j: (j,), memory_space=pltpu.SMEM),
            pl.BlockSpec(memory_space=pl.ANY),
            pl.BlockSpec(memory_space=pl.ANY),
        ],
        out_specs=[
            pl.BlockSpec((CH, EMB), lambda j: (j, 0)),
            pl.BlockSpec((CH, EMB), lambda j: (j, 0)),
        ],
        out_shape=[jax.ShapeDtypeStruct((B, EMB), jnp.float32),
                   jax.ShapeDtypeStruct((B, EMB), jnp.float32)],
        scratch_shapes=[pltpu.SemaphoreType.DMA, pltpu.SemaphoreType.DMA],
    )(u, i, user_table, item_table)


def _mlp_kernel(ue_ref, ie_ref, w1_ref, b1_ref, w2_ref, b2_ref,
                w3_ref, b3_ref, w4_ref, b4_ref, o_ref):
    ue = ue_ref[...]
    ie = ie_ref[...]
    x = (jnp.dot(ue, w1_ref[:EMB, :], preferred_element_type=jnp.float32)
         + jnp.dot(ie, w1_ref[EMB:, :], preferred_element_type=jnp.float32)
         + b1_ref[...])
    x = jnp.maximum(x, 0.0)
    x = jnp.maximum(jnp.dot(x, w2_ref[...], preferred_element_type=jnp.float32)
                    + b2_ref[...], 0.0)
    x = jnp.maximum(jnp.dot(x, w3_ref[...], preferred_element_type=jnp.float32)
                    + b3_ref[...], 0.0)
    o_ref[...] = (jnp.dot(x, w4_ref[...], preferred_element_type=jnp.float32)
                  + b4_ref[...])


@functools.partial(jax.jit, static_argnames=("bm",))
def _tc_mlp(ue, ie, W1, b1, W2, b2, W3, b3, W4, b4, bm=2048):
    nblk = B // bm
    full = lambda shape: pl.BlockSpec(shape, lambda j: tuple(0 for _ in shape))
    return pl.pallas_call(
        _mlp_kernel,
        grid=(nblk,),
        in_specs=[
            pl.BlockSpec((bm, EMB), lambda j: (j, 0)),
            pl.BlockSpec((bm, EMB), lambda j: (j, 0)),
            full(W1.shape), full(b1.shape),
            full(W2.shape), full(b2.shape),
            full(W3.shape), full(b3.shape),
            full(W4.shape), full(b4.shape),
        ],
        out_specs=pl.BlockSpec((bm, 1), lambda j: (j, 0)),
        out_shape=jax.ShapeDtypeStruct((B, 1), jnp.float32),
    )(ue, ie, W1, b1, W2, b2, W3, b3, W4, b4)


def kernel(u, i, user_table, item_table, W1, b1, W2, b2, W3, b3, W4, b4):
    ue, ie = _tc_gather(user_table, item_table,
                        u.astype(jnp.int32), i.astype(jnp.int32))
    out = _tc_mlp(ue, ie,
                  W1, b1.reshape(1, -1), W2, b2.reshape(1, -1),
                  W3, b3.reshape(1, -1), W4, b4.reshape(1, -1))
    return out.reshape(B)
